# Initial kernel scaffold; baseline (speedup 1.0000x reference)
import jax
import jax.numpy as jnp
from jax.experimental import pallas as pl


def kernel(q_curve_xyz, b_curve_xyz, q_mask, b_mask):
    raise NotImplementedError("write your pallas kernel here")


# speedup vs baseline: 10.8014x; 10.8014x over previous
import functools

import jax
import jax.numpy as jnp
from jax.experimental import pallas as pl

T = 16
TOP_K = 3
KNN_K = 16
N_BLK = 128


def _fused_kernel(q_ref, b_ref, qm_ref, bm_ref, dist_ref, ind_ref):
    qx, qy, qz = q_ref[0], q_ref[1], q_ref[2]
    bx, by, bz = b_ref[0], b_ref[1], b_ref[2]
    qm = qm_ref[...]
    bm = bm_ref[...]
    n_blk, m_sz = qx.shape[0], bx.shape[1]

    qsq = qx * qx + qy * qy + qz * qz
    bsq = bx * bx + by * by + bz * bz

    qxr = qx.astype(jnp.bfloat16).astype(jnp.float32)
    qyr = qy.astype(jnp.bfloat16).astype(jnp.float32)
    qzr = qz.astype(jnp.bfloat16).astype(jnp.float32)
    bxr = bx.astype(jnp.bfloat16).astype(jnp.float32)
    byr = by.astype(jnp.bfloat16).astype(jnp.float32)
    bzr = bz.astype(jnp.bfloat16).astype(jnp.float32)

    zeros = jnp.zeros((n_blk, m_sz), jnp.float32)
    m1, m2, m3, cnt = zeros, zeros, zeros, zeros
    for t in range(T):
        qx_t = qxr[:, t:t + 1]
        qy_t = qyr[:, t:t + 1]
        qz_t = qzr[:, t:t + 1]
        bx_t = bxr[t:t + 1, :]
        by_t = byr[t:t + 1, :]
        bz_t = bzr[t:t + 1, :]
        dot = qx_t * bx_t + qy_t * by_t + qz_t * bz_t
        s = (qsq[:, t:t + 1] + bsq[t:t + 1, :]) - 2.0 * dot
        s = jnp.maximum(s, 1e-12)
        mk = qm[:, t:t + 1] * bm[t:t + 1, :]
        s = s * mk
        cnt = cnt + mk
        gt1 = s > m1
        gt2 = s > m2
        gt3 = s > m3
        m3 = jnp.where(gt2, m2, jnp.where(gt3, s, m3))
        m2 = jnp.where(gt1, m1, jnp.where(gt2, s, m2))
        m1 = jnp.where(gt1, s, m1)

    d_sq = jnp.where(cnt <= 5.0, m1, jnp.where(cnt <= 10.0, m2, m3))
    d = jnp.sqrt(d_sq)

    iota = jax.lax.broadcasted_iota(jnp.int32, (n_blk, m_sz), 1)
    dists, inds = [], []
    for _ in range(KNN_K):
        rowmin = jnp.min(d, axis=1, keepdims=True)
        cand = jnp.where(d == rowmin, iota, m_sz)
        amin = jnp.min(cand, axis=1, keepdims=True)
        dists.append(rowmin)
        inds.append(amin)
        d = jnp.where(cand == amin, jnp.inf, d)
    dist_ref[...] = jnp.concatenate(dists, axis=1)
    ind_ref[...] = jnp.concatenate(inds, axis=1)


@jax.jit
def kernel(q_curve_xyz, b_curve_xyz, q_mask, b_mask):
    t, n, _ = q_curve_xyz.shape
    m = b_curve_xyz.shape[1]
    q = jnp.transpose(q_curve_xyz, (2, 1, 0))
    b = jnp.transpose(b_curve_xyz, (2, 0, 1))
    qm = q_mask.astype(jnp.float32).T
    bm = b_mask.astype(jnp.float32)

    grid = (n // N_BLK,)
    dist, ind = pl.pallas_call(
        _fused_kernel,
        grid=grid,
        in_specs=[
            pl.BlockSpec((3, N_BLK, t), lambda i: (0, i, 0)),
            pl.BlockSpec((3, t, m), lambda i: (0, 0, 0)),
            pl.BlockSpec((N_BLK, t), lambda i: (i, 0)),
            pl.BlockSpec((t, m), lambda i: (0, 0)),
        ],
        out_specs=[
            pl.BlockSpec((N_BLK, KNN_K), lambda i: (i, 0)),
            pl.BlockSpec((N_BLK, KNN_K), lambda i: (i, 0)),
        ],
        out_shape=[
            jax.ShapeDtypeStruct((n, KNN_K), jnp.float32),
            jax.ShapeDtypeStruct((n, KNN_K), jnp.int32),
        ],
    )(q, b, qm, bm)
    return dist, ind


# speedup vs baseline: 12.0487x; 1.1155x over previous
import functools

import jax
import jax.numpy as jnp
from jax.experimental import pallas as pl

T = 16
TOP_K = 3
KNN_K = 16
N_BLK = 256


def _fused_kernel(q_ref, b_ref, qm_ref, bm_ref, dist_ref, ind_ref):
    qx, qy, qz = q_ref[0], q_ref[1], q_ref[2]
    bx, by, bz = b_ref[0], b_ref[1], b_ref[2]
    qm = qm_ref[...]
    bm = bm_ref[...]
    n_blk, m_sz = qx.shape[0], bx.shape[1]

    qsq = qx * qx + qy * qy + qz * qz
    bsq = bx * bx + by * by + bz * bz

    qxr = qx.astype(jnp.bfloat16).astype(jnp.float32)
    qyr = qy.astype(jnp.bfloat16).astype(jnp.float32)
    qzr = qz.astype(jnp.bfloat16).astype(jnp.float32)
    bxr = bx.astype(jnp.bfloat16).astype(jnp.float32)
    byr = by.astype(jnp.bfloat16).astype(jnp.float32)
    bzr = bz.astype(jnp.bfloat16).astype(jnp.float32)

    zeros = jnp.zeros((n_blk, m_sz), jnp.float32)
    m1, m2, m3, cnt = zeros, zeros, zeros, zeros
    for t in range(T):
        qx_t = qxr[:, t:t + 1]
        qy_t = qyr[:, t:t + 1]
        qz_t = qzr[:, t:t + 1]
        bx_t = bxr[t:t + 1, :]
        by_t = byr[t:t + 1, :]
        bz_t = bzr[t:t + 1, :]
        dot = qx_t * bx_t + qy_t * by_t + qz_t * bz_t
        s = (qsq[:, t:t + 1] + bsq[t:t + 1, :]) - 2.0 * dot
        s = jnp.maximum(s, 1e-12)
        mk = qm[:, t:t + 1] * bm[t:t + 1, :]
        s = s * mk
        cnt = cnt + mk
        gt1 = s > m1
        gt2 = s > m2
        gt3 = s > m3
        m3 = jnp.where(gt2, m2, jnp.where(gt3, s, m3))
        m2 = jnp.where(gt1, m1, jnp.where(gt2, s, m2))
        m1 = jnp.where(gt1, s, m1)

    d_sq = jnp.where(cnt <= 5.0, m1, jnp.where(cnt <= 10.0, m2, m3))
    d = jnp.sqrt(d_sq)

    iota = jax.lax.broadcasted_iota(jnp.int32, (n_blk, m_sz), 1)
    dists, inds = [], []
    for _ in range(KNN_K):
        rowmin = jnp.min(d, axis=1, keepdims=True)
        cand = jnp.where(d == rowmin, iota, m_sz)
        amin = jnp.min(cand, axis=1, keepdims=True)
        dists.append(rowmin)
        inds.append(amin)
        d = jnp.where(cand == amin, jnp.inf, d)
    dist_ref[...] = jnp.concatenate(dists, axis=1)
    ind_ref[...] = jnp.concatenate(inds, axis=1)


@jax.jit
def kernel(q_curve_xyz, b_curve_xyz, q_mask, b_mask):
    t, n, _ = q_curve_xyz.shape
    m = b_curve_xyz.shape[1]
    q = jnp.transpose(q_curve_xyz, (2, 1, 0))
    b = jnp.transpose(b_curve_xyz, (2, 0, 1))
    qm = q_mask.astype(jnp.float32).T
    bm = b_mask.astype(jnp.float32)

    grid = (n // N_BLK,)
    dist, ind = pl.pallas_call(
        _fused_kernel,
        grid=grid,
        in_specs=[
            pl.BlockSpec((3, N_BLK, t), lambda i: (0, i, 0)),
            pl.BlockSpec((3, t, m), lambda i: (0, 0, 0)),
            pl.BlockSpec((N_BLK, t), lambda i: (i, 0)),
            pl.BlockSpec((t, m), lambda i: (0, 0)),
        ],
        out_specs=[
            pl.BlockSpec((N_BLK, KNN_K), lambda i: (i, 0)),
            pl.BlockSpec((N_BLK, KNN_K), lambda i: (i, 0)),
        ],
        out_shape=[
            jax.ShapeDtypeStruct((n, KNN_K), jnp.float32),
            jax.ShapeDtypeStruct((n, KNN_K), jnp.int32),
        ],
    )(q, b, qm, bm)
    return dist, ind


# speedup vs baseline: 12.8369x; 1.0654x over previous
import functools

import jax
import jax.numpy as jnp
from jax.experimental import pallas as pl

T = 16
TOP_K = 3
KNN_K = 16
N_BLK = 512


def _fused_kernel(q_ref, b_ref, qm_ref, bm_ref, dist_ref, ind_ref):
    qx, qy, qz = q_ref[0], q_ref[1], q_ref[2]
    bx, by, bz = b_ref[0], b_ref[1], b_ref[2]
    qm = qm_ref[...]
    bm = bm_ref[...]
    n_blk, m_sz = qx.shape[0], bx.shape[1]

    qsq = qx * qx + qy * qy + qz * qz
    bsq = bx * bx + by * by + bz * bz

    qxr = qx.astype(jnp.bfloat16).astype(jnp.float32)
    qyr = qy.astype(jnp.bfloat16).astype(jnp.float32)
    qzr = qz.astype(jnp.bfloat16).astype(jnp.float32)
    bxr = bx.astype(jnp.bfloat16).astype(jnp.float32)
    byr = by.astype(jnp.bfloat16).astype(jnp.float32)
    bzr = bz.astype(jnp.bfloat16).astype(jnp.float32)

    zeros = jnp.zeros((n_blk, m_sz), jnp.float32)
    m1, m2, m3, cnt = zeros, zeros, zeros, zeros
    for t in range(T):
        qx_t = qxr[:, t:t + 1]
        qy_t = qyr[:, t:t + 1]
        qz_t = qzr[:, t:t + 1]
        bx_t = bxr[t:t + 1, :]
        by_t = byr[t:t + 1, :]
        bz_t = bzr[t:t + 1, :]
        dot = qx_t * bx_t + qy_t * by_t + qz_t * bz_t
        s = (qsq[:, t:t + 1] + bsq[t:t + 1, :]) - 2.0 * dot
        s = jnp.maximum(s, 1e-12)
        mk = qm[:, t:t + 1] * bm[t:t + 1, :]
        s = s * mk
        cnt = cnt + mk
        gt1 = s > m1
        gt2 = s > m2
        gt3 = s > m3
        m3 = jnp.where(gt2, m2, jnp.where(gt3, s, m3))
        m2 = jnp.where(gt1, m1, jnp.where(gt2, s, m2))
        m1 = jnp.where(gt1, s, m1)

    d_sq = jnp.where(cnt <= 5.0, m1, jnp.where(cnt <= 10.0, m2, m3))
    d = jnp.sqrt(d_sq)

    iota = jax.lax.broadcasted_iota(jnp.int32, (n_blk, m_sz), 1)
    dists, inds = [], []
    for _ in range(KNN_K):
        rowmin = jnp.min(d, axis=1, keepdims=True)
        cand = jnp.where(d == rowmin, iota, m_sz)
        amin = jnp.min(cand, axis=1, keepdims=True)
        dists.append(rowmin)
        inds.append(amin)
        d = jnp.where(cand == amin, jnp.inf, d)
    dist_ref[...] = jnp.concatenate(dists, axis=1)
    ind_ref[...] = jnp.concatenate(inds, axis=1)


@jax.jit
def kernel(q_curve_xyz, b_curve_xyz, q_mask, b_mask):
    t, n, _ = q_curve_xyz.shape
    m = b_curve_xyz.shape[1]
    q = jnp.transpose(q_curve_xyz, (2, 1, 0))
    b = jnp.transpose(b_curve_xyz, (2, 0, 1))
    qm = q_mask.astype(jnp.float32).T
    bm = b_mask.astype(jnp.float32)

    grid = (n // N_BLK,)
    dist, ind = pl.pallas_call(
        _fused_kernel,
        grid=grid,
        in_specs=[
            pl.BlockSpec((3, N_BLK, t), lambda i: (0, i, 0)),
            pl.BlockSpec((3, t, m), lambda i: (0, 0, 0)),
            pl.BlockSpec((N_BLK, t), lambda i: (i, 0)),
            pl.BlockSpec((t, m), lambda i: (0, 0)),
        ],
        out_specs=[
            pl.BlockSpec((N_BLK, KNN_K), lambda i: (i, 0)),
            pl.BlockSpec((N_BLK, KNN_K), lambda i: (i, 0)),
        ],
        out_shape=[
            jax.ShapeDtypeStruct((n, KNN_K), jnp.float32),
            jax.ShapeDtypeStruct((n, KNN_K), jnp.int32),
        ],
    )(q, b, qm, bm)
    return dist, ind


# speedup vs baseline: 13.4419x; 1.0471x over previous
import functools

import jax
import jax.numpy as jnp
from jax.experimental import pallas as pl

T = 16
TOP_K = 3
KNN_K = 16
N_BLK = 512


def _fused_kernel(q_ref, b_ref, qm_ref, bm_ref, dist_ref, ind_ref):
    qx, qy, qz = q_ref[0], q_ref[1], q_ref[2]
    bx, by, bz = b_ref[0], b_ref[1], b_ref[2]
    qm = qm_ref[...]
    bm = bm_ref[...]
    n_blk, m_sz = qx.shape[0], bx.shape[1]

    qsq = qx * qx + qy * qy + qz * qz
    bsq = bx * bx + by * by + bz * bz

    qxr = qx.astype(jnp.bfloat16).astype(jnp.float32)
    qyr = qy.astype(jnp.bfloat16).astype(jnp.float32)
    qzr = qz.astype(jnp.bfloat16).astype(jnp.float32)
    bxr = bx.astype(jnp.bfloat16).astype(jnp.float32)
    byr = by.astype(jnp.bfloat16).astype(jnp.float32)
    bzr = bz.astype(jnp.bfloat16).astype(jnp.float32)

    zeros = jnp.zeros((n_blk, m_sz), jnp.float32)
    m1, m2, m3, cnt = zeros, zeros, zeros, zeros
    for t in range(T):
        qx_t = qxr[:, t:t + 1]
        qy_t = qyr[:, t:t + 1]
        qz_t = qzr[:, t:t + 1]
        bx_t = bxr[t:t + 1, :]
        by_t = byr[t:t + 1, :]
        bz_t = bzr[t:t + 1, :]
        dot = qx_t * bx_t + qy_t * by_t + qz_t * bz_t
        s = (qsq[:, t:t + 1] + bsq[t:t + 1, :]) - 2.0 * dot
        s = jnp.maximum(s, 1e-12)
        mk = qm[:, t:t + 1] * bm[t:t + 1, :]
        s = s * mk
        cnt = cnt + mk
        m3 = jnp.maximum(m3, jnp.minimum(s, m2))
        m2 = jnp.maximum(m2, jnp.minimum(s, m1))
        m1 = jnp.maximum(m1, s)

    d_sq = jnp.where(cnt <= 5.0, m1, jnp.where(cnt <= 10.0, m2, m3))
    d = jnp.sqrt(d_sq)

    iota = jax.lax.broadcasted_iota(jnp.int32, (n_blk, m_sz), 1)
    dists, inds = [], []
    for _ in range(KNN_K):
        rowmin = jnp.min(d, axis=1, keepdims=True)
        cand = jnp.where(d == rowmin, iota, m_sz)
        amin = jnp.min(cand, axis=1, keepdims=True)
        dists.append(rowmin)
        inds.append(amin)
        d = jnp.where(cand == amin, jnp.inf, d)
    dist_ref[...] = jnp.concatenate(dists, axis=1)
    ind_ref[...] = jnp.concatenate(inds, axis=1)


@jax.jit
def kernel(q_curve_xyz, b_curve_xyz, q_mask, b_mask):
    t, n, _ = q_curve_xyz.shape
    m = b_curve_xyz.shape[1]
    q = jnp.transpose(q_curve_xyz, (2, 1, 0))
    b = jnp.transpose(b_curve_xyz, (2, 0, 1))
    qm = q_mask.astype(jnp.float32).T
    bm = b_mask.astype(jnp.float32)

    grid = (n // N_BLK,)
    dist, ind = pl.pallas_call(
        _fused_kernel,
        grid=grid,
        in_specs=[
            pl.BlockSpec((3, N_BLK, t), lambda i: (0, i, 0)),
            pl.BlockSpec((3, t, m), lambda i: (0, 0, 0)),
            pl.BlockSpec((N_BLK, t), lambda i: (i, 0)),
            pl.BlockSpec((t, m), lambda i: (0, 0)),
        ],
        out_specs=[
            pl.BlockSpec((N_BLK, KNN_K), lambda i: (i, 0)),
            pl.BlockSpec((N_BLK, KNN_K), lambda i: (i, 0)),
        ],
        out_shape=[
            jax.ShapeDtypeStruct((n, KNN_K), jnp.float32),
            jax.ShapeDtypeStruct((n, KNN_K), jnp.int32),
        ],
    )(q, b, qm, bm)
    return dist, ind


# speedup vs baseline: 13.9095x; 1.0348x over previous
import functools

import jax
import jax.numpy as jnp
from jax.experimental import pallas as pl

T = 16
TOP_K = 3
KNN_K = 16
N_BLK = 512


def _fused_kernel(q_ref, b_ref, qm_ref, bm_ref, dist_ref, ind_ref):
    qx, qy, qz = q_ref[0], q_ref[1], q_ref[2]
    bx, by, bz = b_ref[0], b_ref[1], b_ref[2]
    qm = qm_ref[...]
    bm = bm_ref[...]
    n_blk, m_sz = qx.shape[0], bx.shape[1]

    qsq = qx * qx + qy * qy + qz * qz
    bsq = bx * bx + by * by + bz * bz

    qxr = qx.astype(jnp.bfloat16).astype(jnp.float32)
    qyr = qy.astype(jnp.bfloat16).astype(jnp.float32)
    qzr = qz.astype(jnp.bfloat16).astype(jnp.float32)
    bxr = bx.astype(jnp.bfloat16).astype(jnp.float32)
    byr = by.astype(jnp.bfloat16).astype(jnp.float32)
    bzr = bz.astype(jnp.bfloat16).astype(jnp.float32)

    zeros = jnp.zeros((n_blk, m_sz), jnp.float32)
    m1, m2, m3, cnt = zeros, zeros, zeros, zeros
    for t in range(T):
        qx_t = qxr[:, t:t + 1]
        qy_t = qyr[:, t:t + 1]
        qz_t = qzr[:, t:t + 1]
        bx_t = bxr[t:t + 1, :]
        by_t = byr[t:t + 1, :]
        bz_t = bzr[t:t + 1, :]
        dot = qx_t * bx_t + qy_t * by_t + qz_t * bz_t
        s = (qsq[:, t:t + 1] + bsq[t:t + 1, :]) - 2.0 * dot
        s = jnp.maximum(s, 1e-12)
        mk = qm[:, t:t + 1] * bm[t:t + 1, :]
        s = s * mk
        cnt = cnt + mk
        m3 = jnp.maximum(m3, jnp.minimum(s, m2))
        m2 = jnp.maximum(m2, jnp.minimum(s, m1))
        m1 = jnp.maximum(m1, s)

    d_sq = jnp.where(cnt <= 5.0, m1, jnp.where(cnt <= 10.0, m2, m3))
    d = jnp.sqrt(d_sq)

    iota = jax.lax.broadcasted_iota(
        jnp.int32, (n_blk, m_sz), 1).astype(jnp.float32)
    dists, inds = [], []
    for _ in range(KNN_K):
        rowmin = jnp.min(d, axis=1, keepdims=True)
        cand = jnp.where(d == rowmin, iota, float(m_sz))
        amin = jnp.min(cand, axis=1, keepdims=True)
        dists.append(rowmin)
        inds.append(amin)
        d = jnp.where(cand == amin, jnp.inf, d)
    dist_ref[...] = jnp.concatenate(dists, axis=1)
    ind_ref[...] = jnp.concatenate(inds, axis=1).astype(jnp.int32)


@jax.jit
def kernel(q_curve_xyz, b_curve_xyz, q_mask, b_mask):
    t, n, _ = q_curve_xyz.shape
    m = b_curve_xyz.shape[1]
    q = jnp.transpose(q_curve_xyz, (2, 1, 0))
    b = jnp.transpose(b_curve_xyz, (2, 0, 1))
    qm = q_mask.astype(jnp.float32).T
    bm = b_mask.astype(jnp.float32)

    grid = (n // N_BLK,)
    dist, ind = pl.pallas_call(
        _fused_kernel,
        grid=grid,
        in_specs=[
            pl.BlockSpec((3, N_BLK, t), lambda i: (0, i, 0)),
            pl.BlockSpec((3, t, m), lambda i: (0, 0, 0)),
            pl.BlockSpec((N_BLK, t), lambda i: (i, 0)),
            pl.BlockSpec((t, m), lambda i: (0, 0)),
        ],
        out_specs=[
            pl.BlockSpec((N_BLK, KNN_K), lambda i: (i, 0)),
            pl.BlockSpec((N_BLK, KNN_K), lambda i: (i, 0)),
        ],
        out_shape=[
            jax.ShapeDtypeStruct((n, KNN_K), jnp.float32),
            jax.ShapeDtypeStruct((n, KNN_K), jnp.int32),
        ],
    )(q, b, qm, bm)
    return dist, ind


# speedup vs baseline: 16.3212x; 1.1734x over previous
import functools

import jax
import jax.numpy as jnp
from jax.experimental import pallas as pl

T = 16
TOP_K = 3
KNN_K = 16
N_BLK = 512


def _fused_kernel(q_ref, b_ref, qm_ref, bm_ref, dist_ref, ind_ref):
    qx, qy, qz = q_ref[0], q_ref[1], q_ref[2]
    bx, by, bz = b_ref[0], b_ref[1], b_ref[2]
    qm = qm_ref[...]
    bm = bm_ref[...]
    n_blk, m_sz = qx.shape[0], bx.shape[1]

    qsq = qx * qx + qy * qy + qz * qz
    bsq = bx * bx + by * by + bz * bz

    qxr = qx.astype(jnp.bfloat16).astype(jnp.float32)
    qyr = qy.astype(jnp.bfloat16).astype(jnp.float32)
    qzr = qz.astype(jnp.bfloat16).astype(jnp.float32)
    bxr = bx.astype(jnp.bfloat16).astype(jnp.float32)
    byr = by.astype(jnp.bfloat16).astype(jnp.float32)
    bzr = bz.astype(jnp.bfloat16).astype(jnp.float32)

    qs_p = qsq + (qm - 1.0) * 1e30
    bs_p = bsq + (bm - 1.0) * 1e30
    cnt = jnp.dot(qm.astype(jnp.bfloat16), bm.astype(jnp.bfloat16),
                  preferred_element_type=jnp.float32)

    sent = jnp.float32(1e-12)
    m1 = jnp.full((n_blk, m_sz), sent, jnp.float32)
    m2, m3 = m1, m1
    for t in range(T):
        qx_t = qxr[:, t:t + 1]
        qy_t = qyr[:, t:t + 1]
        qz_t = qzr[:, t:t + 1]
        bx_t = bxr[t:t + 1, :]
        by_t = byr[t:t + 1, :]
        bz_t = bzr[t:t + 1, :]
        dot = qx_t * bx_t + qy_t * by_t + qz_t * bz_t
        s = (qs_p[:, t:t + 1] + bs_p[t:t + 1, :]) - 2.0 * dot
        s = jnp.maximum(s, sent)
        m3 = jnp.maximum(m3, jnp.minimum(s, m2))
        m2 = jnp.maximum(m2, jnp.minimum(s, m1))
        m1 = jnp.maximum(m1, s)

    d_sq = jnp.where(cnt <= 5.0, m1, jnp.where(cnt <= 10.0, m2, m3))
    d = jnp.where(cnt == 0.0, 0.0, jnp.sqrt(d_sq))

    iota = jax.lax.broadcasted_iota(
        jnp.int32, (n_blk, m_sz), 1).astype(jnp.float32)
    dists, inds = [], []
    for _ in range(KNN_K):
        rowmin = jnp.min(d, axis=1, keepdims=True)
        cand = jnp.where(d == rowmin, iota, float(m_sz))
        amin = jnp.min(cand, axis=1, keepdims=True)
        dists.append(rowmin)
        inds.append(amin)
        d = jnp.where(cand == amin, jnp.inf, d)
    dist_ref[...] = jnp.concatenate(dists, axis=1)
    ind_ref[...] = jnp.concatenate(inds, axis=1).astype(jnp.int32)


@jax.jit
def kernel(q_curve_xyz, b_curve_xyz, q_mask, b_mask):
    t, n, _ = q_curve_xyz.shape
    m = b_curve_xyz.shape[1]
    q = jnp.transpose(q_curve_xyz, (2, 1, 0))
    b = jnp.transpose(b_curve_xyz, (2, 0, 1))
    qm = q_mask.astype(jnp.float32).T
    bm = b_mask.astype(jnp.float32)

    grid = (n // N_BLK,)
    dist, ind = pl.pallas_call(
        _fused_kernel,
        grid=grid,
        in_specs=[
            pl.BlockSpec((3, N_BLK, t), lambda i: (0, i, 0)),
            pl.BlockSpec((3, t, m), lambda i: (0, 0, 0)),
            pl.BlockSpec((N_BLK, t), lambda i: (i, 0)),
            pl.BlockSpec((t, m), lambda i: (0, 0)),
        ],
        out_specs=[
            pl.BlockSpec((N_BLK, KNN_K), lambda i: (i, 0)),
            pl.BlockSpec((N_BLK, KNN_K), lambda i: (i, 0)),
        ],
        out_shape=[
            jax.ShapeDtypeStruct((n, KNN_K), jnp.float32),
            jax.ShapeDtypeStruct((n, KNN_K), jnp.int32),
        ],
    )(q, b, qm, bm)
    return dist, ind


# speedup vs baseline: 16.6472x; 1.0200x over previous
import functools

import jax
import jax.numpy as jnp
from jax.experimental import pallas as pl

T = 16
TOP_K = 3
KNN_K = 16
N_BLK = 512


def _fused_kernel(q_ref, b_ref, qm_ref, bm_ref, dist_ref, ind_ref):
    qx, qy, qz = q_ref[0], q_ref[1], q_ref[2]
    bx, by, bz = b_ref[0], b_ref[1], b_ref[2]
    qm = qm_ref[...]
    bm = bm_ref[...]
    n_blk, m_sz = qx.shape[0], bx.shape[1]

    qsq = qx * qx + qy * qy + qz * qz
    bsq = bx * bx + by * by + bz * bz

    qxr = qx.astype(jnp.bfloat16).astype(jnp.float32)
    qyr = qy.astype(jnp.bfloat16).astype(jnp.float32)
    qzr = qz.astype(jnp.bfloat16).astype(jnp.float32)
    bxr = bx.astype(jnp.bfloat16).astype(jnp.float32)
    byr = by.astype(jnp.bfloat16).astype(jnp.float32)
    bzr = bz.astype(jnp.bfloat16).astype(jnp.float32)

    qs_p = qsq + (qm - 1.0) * 1e30
    bs_p = bsq + (bm - 1.0) * 1e30
    cnt = jnp.dot(qm.astype(jnp.bfloat16), bm.astype(jnp.bfloat16),
                  preferred_element_type=jnp.float32)

    sent = jnp.float32(1e-12)
    m1 = m2 = m3 = None
    for t in range(T):
        qx_t = qxr[:, t:t + 1]
        qy_t = qyr[:, t:t + 1]
        qz_t = qzr[:, t:t + 1]
        bx_t = bxr[t:t + 1, :]
        by_t = byr[t:t + 1, :]
        bz_t = bzr[t:t + 1, :]
        dot = qx_t * bx_t + qy_t * by_t + qz_t * bz_t
        s = (qs_p[:, t:t + 1] + bs_p[t:t + 1, :]) - 2.0 * dot
        s = jnp.maximum(s, sent)
        if t == 0:
            m1 = s
        elif t == 1:
            m2 = jnp.minimum(s, m1)
            m1 = jnp.maximum(m1, s)
        elif t == 2:
            m3 = jnp.minimum(s, m2)
            m2 = jnp.maximum(m2, jnp.minimum(s, m1))
            m1 = jnp.maximum(m1, s)
        else:
            m3 = jnp.maximum(m3, jnp.minimum(s, m2))
            m2 = jnp.maximum(m2, jnp.minimum(s, m1))
            m1 = jnp.maximum(m1, s)

    d_sq = jnp.where(cnt <= 5.0, m1, jnp.where(cnt <= 10.0, m2, m3))
    d = jnp.where(cnt == 0.0, 0.0, jnp.sqrt(d_sq))

    iota = jax.lax.broadcasted_iota(
        jnp.int32, (n_blk, m_sz), 1).astype(jnp.float32)
    dists, inds = [], []
    for _ in range(KNN_K):
        rowmin = jnp.min(d, axis=1, keepdims=True)
        cand = jnp.where(d == rowmin, iota, float(m_sz))
        amin = jnp.min(cand, axis=1, keepdims=True)
        dists.append(rowmin)
        inds.append(amin)
        d = jnp.where(cand == amin, jnp.inf, d)
    dist_ref[...] = jnp.concatenate(dists, axis=1)
    ind_ref[...] = jnp.concatenate(inds, axis=1).astype(jnp.int32)


@jax.jit
def kernel(q_curve_xyz, b_curve_xyz, q_mask, b_mask):
    t, n, _ = q_curve_xyz.shape
    m = b_curve_xyz.shape[1]
    q = jnp.transpose(q_curve_xyz, (2, 1, 0))
    b = jnp.transpose(b_curve_xyz, (2, 0, 1))
    qm = q_mask.astype(jnp.float32).T
    bm = b_mask.astype(jnp.float32)

    grid = (n // N_BLK,)
    dist, ind = pl.pallas_call(
        _fused_kernel,
        grid=grid,
        in_specs=[
            pl.BlockSpec((3, N_BLK, t), lambda i: (0, i, 0)),
            pl.BlockSpec((3, t, m), lambda i: (0, 0, 0)),
            pl.BlockSpec((N_BLK, t), lambda i: (i, 0)),
            pl.BlockSpec((t, m), lambda i: (0, 0)),
        ],
        out_specs=[
            pl.BlockSpec((N_BLK, KNN_K), lambda i: (i, 0)),
            pl.BlockSpec((N_BLK, KNN_K), lambda i: (i, 0)),
        ],
        out_shape=[
            jax.ShapeDtypeStruct((n, KNN_K), jnp.float32),
            jax.ShapeDtypeStruct((n, KNN_K), jnp.int32),
        ],
    )(q, b, qm, bm)
    return dist, ind


# speedup vs baseline: 17.2597x; 1.0368x over previous
import functools

import jax
import jax.numpy as jnp
from jax.experimental import pallas as pl

T = 16
TOP_K = 3
KNN_K = 16
N_BLK = 512


def _fused_kernel(q_ref, b_ref, qm_ref, bm_ref, dist_ref, ind_ref):
    qx, qy, qz = q_ref[0], q_ref[1], q_ref[2]
    bx, by, bz = b_ref[0], b_ref[1], b_ref[2]
    qm = qm_ref[...]
    bm = bm_ref[...]
    n_blk, m_sz = qx.shape[0], bx.shape[1]

    qsq = qx * qx + qy * qy + qz * qz
    bsq = bx * bx + by * by + bz * bz

    qxr = qx.astype(jnp.bfloat16).astype(jnp.float32)
    qyr = qy.astype(jnp.bfloat16).astype(jnp.float32)
    qzr = qz.astype(jnp.bfloat16).astype(jnp.float32)
    bxr = bx.astype(jnp.bfloat16).astype(jnp.float32)
    byr = by.astype(jnp.bfloat16).astype(jnp.float32)
    bzr = bz.astype(jnp.bfloat16).astype(jnp.float32)

    qs_p = qsq + (qm - 1.0) * 1e30
    bs_p = bsq + (bm - 1.0) * 1e30
    cnt = jnp.dot(qm.astype(jnp.bfloat16), bm.astype(jnp.bfloat16),
                  preferred_element_type=jnp.float32)

    sent = jnp.float32(1e-12)
    m1 = m2 = m3 = None
    for t in range(T):
        qx_t = qxr[:, t:t + 1]
        qy_t = qyr[:, t:t + 1]
        qz_t = qzr[:, t:t + 1]
        bx_t = bxr[t:t + 1, :]
        by_t = byr[t:t + 1, :]
        bz_t = bzr[t:t + 1, :]
        dot = qx_t * bx_t + qy_t * by_t + qz_t * bz_t
        s = (qs_p[:, t:t + 1] + bs_p[t:t + 1, :]) - 2.0 * dot
        if t == 0:
            m1 = s
        elif t == 1:
            m2 = jnp.minimum(s, m1)
            m1 = jnp.maximum(m1, s)
        elif t == 2:
            m3 = jnp.minimum(s, m2)
            m2 = jnp.maximum(m2, jnp.minimum(s, m1))
            m1 = jnp.maximum(m1, s)
        else:
            m3 = jnp.maximum(m3, jnp.minimum(s, m2))
            m2 = jnp.maximum(m2, jnp.minimum(s, m1))
            m1 = jnp.maximum(m1, s)

    d_sq = jnp.where(cnt <= 5.0, m1, jnp.where(cnt <= 10.0, m2, m3))
    d = jnp.where(cnt == 0.0, 0.0,
                  jnp.sqrt(jnp.maximum(d_sq, sent)))

    iota = jax.lax.broadcasted_iota(
        jnp.int32, (n_blk, m_sz), 1).astype(jnp.float32)
    dists, inds = [], []
    for _ in range(KNN_K):
        rowmin = jnp.min(d, axis=1, keepdims=True)
        cand = jnp.where(d == rowmin, iota, float(m_sz))
        amin = jnp.min(cand, axis=1, keepdims=True)
        dists.append(rowmin)
        inds.append(amin)
        d = jnp.where(cand == amin, jnp.inf, d)
    dist_ref[...] = jnp.concatenate(dists, axis=1)
    ind_ref[...] = jnp.concatenate(inds, axis=1).astype(jnp.int32)


@jax.jit
def kernel(q_curve_xyz, b_curve_xyz, q_mask, b_mask):
    t, n, _ = q_curve_xyz.shape
    m = b_curve_xyz.shape[1]
    q = jnp.transpose(q_curve_xyz, (2, 1, 0))
    b = jnp.transpose(b_curve_xyz, (2, 0, 1))
    qm = q_mask.astype(jnp.float32).T
    bm = b_mask.astype(jnp.float32)

    grid = (n // N_BLK,)
    dist, ind = pl.pallas_call(
        _fused_kernel,
        grid=grid,
        in_specs=[
            pl.BlockSpec((3, N_BLK, t), lambda i: (0, i, 0)),
            pl.BlockSpec((3, t, m), lambda i: (0, 0, 0)),
            pl.BlockSpec((N_BLK, t), lambda i: (i, 0)),
            pl.BlockSpec((t, m), lambda i: (0, 0)),
        ],
        out_specs=[
            pl.BlockSpec((N_BLK, KNN_K), lambda i: (i, 0)),
            pl.BlockSpec((N_BLK, KNN_K), lambda i: (i, 0)),
        ],
        out_shape=[
            jax.ShapeDtypeStruct((n, KNN_K), jnp.float32),
            jax.ShapeDtypeStruct((n, KNN_K), jnp.int32),
        ],
    )(q, b, qm, bm)
    return dist, ind


# speedup vs baseline: 17.2647x; 1.0003x over previous
import functools

import jax
import jax.numpy as jnp
from jax.experimental import pallas as pl

T = 16
TOP_K = 3
KNN_K = 16
N_BLK = 512


def _fused_kernel(q_ref, b_ref, qm_ref, bm_ref, dist_ref, ind_ref):
    qx, qy, qz = q_ref[0], q_ref[1], q_ref[2]
    bx, by, bz = b_ref[0], b_ref[1], b_ref[2]
    qm = qm_ref[...]
    bm = bm_ref[...]
    n_blk, m_sz = qx.shape[0], bx.shape[1]

    qsq = qx * qx + qy * qy + qz * qz
    bsq = bx * bx + by * by + bz * bz

    qxr = qx.astype(jnp.bfloat16).astype(jnp.float32)
    qyr = qy.astype(jnp.bfloat16).astype(jnp.float32)
    qzr = qz.astype(jnp.bfloat16).astype(jnp.float32)
    bxr = bx.astype(jnp.bfloat16).astype(jnp.float32)
    byr = by.astype(jnp.bfloat16).astype(jnp.float32)
    bzr = bz.astype(jnp.bfloat16).astype(jnp.float32)

    qs_p = qsq + (qm - 1.0) * 1e30
    bs_p = bsq + (bm - 1.0) * 1e30
    cnt = jnp.dot(qm.astype(jnp.bfloat16), bm.astype(jnp.bfloat16),
                  preferred_element_type=jnp.float32)

    sent = jnp.float32(1e-12)
    m1 = m2 = m3 = None
    for t in range(T):
        qx_t = qxr[:, t:t + 1]
        qy_t = qyr[:, t:t + 1]
        qz_t = qzr[:, t:t + 1]
        bx_t = bxr[t:t + 1, :]
        by_t = byr[t:t + 1, :]
        bz_t = bzr[t:t + 1, :]
        dot = qx_t * bx_t + qy_t * by_t + qz_t * bz_t
        s = (qs_p[:, t:t + 1] + bs_p[t:t + 1, :]) - 2.0 * dot
        if t == 0:
            m1 = s
        elif t == 1:
            m2 = jnp.minimum(s, m1)
            m1 = jnp.maximum(m1, s)
        elif t == 2:
            m3 = jnp.minimum(s, m2)
            m2 = jnp.maximum(m2, jnp.minimum(s, m1))
            m1 = jnp.maximum(m1, s)
        else:
            m3 = jnp.maximum(m3, jnp.minimum(s, m2))
            m2 = jnp.maximum(m2, jnp.minimum(s, m1))
            m1 = jnp.maximum(m1, s)

    d_sq = jnp.where(cnt <= 5.0, m1, jnp.where(cnt <= 10.0, m2, m3))
    d = jnp.where(cnt == 0.0, 0.0,
                  jnp.sqrt(jnp.maximum(d_sq, sent)))

    iota = jax.lax.broadcasted_iota(
        jnp.int32, (n_blk, m_sz), 1).astype(jnp.float32)

    def _rowmin(x):
        q = m_sz // 4
        folded = jnp.minimum(jnp.minimum(x[:, :q], x[:, q:2 * q]),
                             jnp.minimum(x[:, 2 * q:3 * q], x[:, 3 * q:]))
        return jnp.min(folded, axis=1, keepdims=True)

    dists, inds = [], []
    for _ in range(KNN_K):
        rowmin = _rowmin(d)
        cand = jnp.where(d == rowmin, iota, float(m_sz))
        amin = _rowmin(cand)
        dists.append(rowmin)
        inds.append(amin)
        d = jnp.where(cand == amin, jnp.inf, d)
    dist_ref[...] = jnp.concatenate(dists, axis=1)
    ind_ref[...] = jnp.concatenate(inds, axis=1).astype(jnp.int32)


@jax.jit
def kernel(q_curve_xyz, b_curve_xyz, q_mask, b_mask):
    t, n, _ = q_curve_xyz.shape
    m = b_curve_xyz.shape[1]
    q = jnp.transpose(q_curve_xyz, (2, 1, 0))
    b = jnp.transpose(b_curve_xyz, (2, 0, 1))
    qm = q_mask.astype(jnp.float32).T
    bm = b_mask.astype(jnp.float32)

    grid = (n // N_BLK,)
    dist, ind = pl.pallas_call(
        _fused_kernel,
        grid=grid,
        in_specs=[
            pl.BlockSpec((3, N_BLK, t), lambda i: (0, i, 0)),
            pl.BlockSpec((3, t, m), lambda i: (0, 0, 0)),
            pl.BlockSpec((N_BLK, t), lambda i: (i, 0)),
            pl.BlockSpec((t, m), lambda i: (0, 0)),
        ],
        out_specs=[
            pl.BlockSpec((N_BLK, KNN_K), lambda i: (i, 0)),
            pl.BlockSpec((N_BLK, KNN_K), lambda i: (i, 0)),
        ],
        out_shape=[
            jax.ShapeDtypeStruct((n, KNN_K), jnp.float32),
            jax.ShapeDtypeStruct((n, KNN_K), jnp.int32),
        ],
    )(q, b, qm, bm)
    return dist, ind


# speedup vs baseline: 17.2679x; 1.0002x over previous
import functools

import jax
import jax.numpy as jnp
from jax.experimental import pallas as pl

T = 16
TOP_K = 3
KNN_K = 16
N_BLK = 512


def _fused_kernel(q_ref, b_ref, qm_ref, bm_ref, dist_ref, ind_ref):
    qx, qy, qz = q_ref[0], q_ref[1], q_ref[2]
    bx, by, bz = b_ref[0], b_ref[1], b_ref[2]
    qm = qm_ref[...]
    bm = bm_ref[...]
    n_blk, m_sz = qx.shape[0], bx.shape[1]

    qsq = qx * qx + qy * qy + qz * qz
    bsq = bx * bx + by * by + bz * bz

    qxr = qx.astype(jnp.bfloat16).astype(jnp.float32)
    qyr = qy.astype(jnp.bfloat16).astype(jnp.float32)
    qzr = qz.astype(jnp.bfloat16).astype(jnp.float32)
    bxr = bx.astype(jnp.bfloat16).astype(jnp.float32)
    byr = by.astype(jnp.bfloat16).astype(jnp.float32)
    bzr = bz.astype(jnp.bfloat16).astype(jnp.float32)

    qs_p = qsq + (qm - 1.0) * 1e30
    bs_p = bsq + (bm - 1.0) * 1e30
    cnt = jnp.dot(qm.astype(jnp.bfloat16), bm.astype(jnp.bfloat16),
                  preferred_element_type=jnp.float32)

    sent = jnp.float32(1e-12)
    m1 = m2 = m3 = None
    for t in range(T):
        qx_t = qxr[:, t:t + 1]
        qy_t = qyr[:, t:t + 1]
        qz_t = qzr[:, t:t + 1]
        bx_t = bxr[t:t + 1, :]
        by_t = byr[t:t + 1, :]
        bz_t = bzr[t:t + 1, :]
        dot = qx_t * bx_t + qy_t * by_t + qz_t * bz_t
        s = (qs_p[:, t:t + 1] + bs_p[t:t + 1, :]) - 2.0 * dot
        if t == 0:
            m1 = s
        elif t == 1:
            m2 = jnp.minimum(s, m1)
            m1 = jnp.maximum(m1, s)
        elif t == 2:
            m3 = jnp.minimum(s, m2)
            m2 = jnp.maximum(m2, jnp.minimum(s, m1))
            m1 = jnp.maximum(m1, s)
        else:
            m3 = jnp.maximum(m3, jnp.minimum(s, m2))
            m2 = jnp.maximum(m2, jnp.minimum(s, m1))
            m1 = jnp.maximum(m1, s)

    d_sq = jnp.where(cnt <= 5.0, m1, jnp.where(cnt <= 10.0, m2, m3))
    d = jnp.where(cnt == 0.0, 0.0,
                  jnp.sqrt(jnp.maximum(d_sq, sent)))

    iota = jax.lax.broadcasted_iota(
        jnp.int32, (n_blk, m_sz), 1).astype(jnp.float32)
    dists, inds = [], []
    for _ in range(KNN_K):
        rowmin = jnp.min(d, axis=1, keepdims=True)
        cand = jnp.where(d == rowmin, iota, float(m_sz))
        amin = jnp.min(cand, axis=1, keepdims=True)
        dists.append(rowmin)
        inds.append(amin)
        d = jnp.where(cand == amin, jnp.inf, d)
    dist_ref[...] = jnp.concatenate(dists, axis=1)
    ind_ref[...] = jnp.concatenate(inds, axis=1).astype(jnp.int32)


@jax.jit
def kernel(q_curve_xyz, b_curve_xyz, q_mask, b_mask):
    t, n, _ = q_curve_xyz.shape
    m = b_curve_xyz.shape[1]
    q = jnp.transpose(q_curve_xyz, (2, 1, 0))
    b = jnp.transpose(b_curve_xyz, (2, 0, 1))
    qm = q_mask.astype(jnp.float32).T
    bm = b_mask.astype(jnp.float32)

    grid = (n // N_BLK,)
    dist, ind = pl.pallas_call(
        _fused_kernel,
        grid=grid,
        in_specs=[
            pl.BlockSpec((3, N_BLK, t), lambda i: (0, i, 0)),
            pl.BlockSpec((3, t, m), lambda i: (0, 0, 0)),
            pl.BlockSpec((N_BLK, t), lambda i: (i, 0)),
            pl.BlockSpec((t, m), lambda i: (0, 0)),
        ],
        out_specs=[
            pl.BlockSpec((N_BLK, KNN_K), lambda i: (i, 0)),
            pl.BlockSpec((N_BLK, KNN_K), lambda i: (i, 0)),
        ],
        out_shape=[
            jax.ShapeDtypeStruct((n, KNN_K), jnp.float32),
            jax.ShapeDtypeStruct((n, KNN_K), jnp.int32),
        ],
    )(q, b, qm, bm)
    return dist, ind
